# baseline (speedup 1.0000x reference)
import functools

import jax
import jax.numpy as jnp
from jax import lax
from jax.experimental import pallas as pl
from jax.experimental.pallas import tpu as pltpu
from jax.experimental.pallas import tpu_sc as plsc

L = 16
SB = 8
RC = 2048


def _tc_relayout_body(x_ref, o_ref):
    ii = jax.lax.broadcasted_iota(jnp.int32, (64, 64), 0)
    jj = jax.lax.broadcasted_iota(jnp.int32, (64, 64), 1)
    eye = (ii == jj).astype(jnp.float32)
    o_ref[...] = jax.lax.dot_general(
        x_ref[...], eye, (((0,), (0,)), ((), ())),
        preferred_element_type=jnp.float32)


def _tc_relayout(xt, n, h):
    return pl.pallas_call(
        _tc_relayout_body,
        grid=(pl.cdiv(n, RC),),
        in_specs=[pl.BlockSpec((h, RC), lambda i: (0, i))],
        out_specs=pl.BlockSpec((RC, h), lambda i: (i, 0)),
        out_shape=jax.ShapeDtypeStruct((n, h), jnp.float32),
    )(xt)


def kernel(user_ids, item_ids, user_emb, item_emb, user_bias, item_bias, W, b):
    B = user_ids.shape[0]
    NU = user_emb.shape[0]
    NI = item_emb.shape[0]
    H = user_emb.shape[1]
    info = plsc.get_sparse_core_info()
    NC, NS = info.num_cores, info.num_subcores
    NW = NC * NS
    bpw = B // NW
    CH = 128
    nchk = bpw // CH

    uids = user_ids.astype(jnp.int32)
    iids = item_ids.astype(jnp.int32)
    ue3 = _tc_relayout(user_emb.T, NU, H).reshape(NU // SB, SB, H)
    ie3 = item_emb.reshape(NI // SB, SB, H)
    ub_tbl = user_bias.reshape(-1)
    ib_tbl = item_bias.reshape(-1)
    wvec = jnp.concatenate(
        [W.reshape(-1), b.reshape(-1), jnp.zeros((6,), jnp.float32)]
    )

    mesh = plsc.VectorSubcoreMesh(core_axis_name="c", subcore_axis_name="s")

    @functools.partial(
        pl.kernel,
        out_type=jax.ShapeDtypeStruct((B,), jnp.float32),
        mesh=mesh,
        scratch_types=[
            pltpu.VMEM((bpw,), jnp.int32),
            pltpu.VMEM((bpw,), jnp.int32),
            pltpu.VMEM((2, L, SB, H), jnp.float32),
            pltpu.VMEM((2, L, SB, H), jnp.float32),
            pltpu.VMEM((bpw,), jnp.float32),
            pltpu.VMEM((bpw,), jnp.float32),
            pltpu.VMEM((bpw,), jnp.float32),
            pltpu.VMEM((H + 8,), jnp.float32),
            pltpu.SemaphoreType.DMA,
            pltpu.SemaphoreType.DMA,
            pltpu.SemaphoreType.DMA,
        ],
        compiler_params=pltpu.CompilerParams(
            needs_layout_passes=False, use_tc_tiling_on_sc=True
        ),
    )
    def biased_gmf_sc(uids_hbm, iids_hbm, ue_hbm, ie_hbm, ub_hbm, ib_hbm,
                      w_hbm, out_hbm,
                      ubi, ibi, ubuf, ibuf, ubv, ibv, outv, wv,
                      sem0, sem1, sem2):
        wid = lax.axis_index("s") * NC + lax.axis_index("c")
        ids = pl.ds(wid * bpw, bpw)

        pltpu.sync_copy(uids_hbm.at[ids], ubi)
        pltpu.sync_copy(iids_hbm.at[ids], ibi)
        pltpu.sync_copy(w_hbm, wv)

        bias_cps = []
        for c in range(nchk):
            rows = pl.ds(c * CH, CH)
            bias_cps.append(
                pltpu.async_copy(ub_hbm.at[ubi.at[rows]], ubv.at[rows], sem2))
            bias_cps.append(
                pltpu.async_copy(ib_hbm.at[ibi.at[rows]], ibv.at[rows], sem2))

        sems = (sem0, sem1)

        def issue(c, buf):
            uv = ubi[pl.ds(c * L, L)] >> 3
            iv = ibi[pl.ds(c * L, L)] >> 3
            for j in range(L):
                pltpu.async_copy(ue_hbm.at[uv[j]], ubuf.at[buf, j], sems[buf])
                pltpu.async_copy(ie_hbm.at[iv[j]], ibuf.at[buf, j], sems[buf])

        def wait(buf):
            for j in range(L):
                pltpu.make_async_copy(ue_hbm.at[0], ubuf.at[buf, j],
                                      sems[buf]).wait()
                pltpu.make_async_copy(ie_hbm.at[0], ibuf.at[buf, j],
                                      sems[buf]).wait()

        wc = [wv[pl.ds(k * L, L)] for k in range(H // L)]
        wtail = wv[pl.ds(H - 8, L)]
        w_bias = wtail[8]
        b_lin = wtail[9]
        lane = lax.iota(jnp.int32, L)

        def compute(c, buf):
            r0 = c * L
            su = ubi[pl.ds(r0, L)] & (SB - 1)
            si = ibi[pl.ds(r0, L)] & (SB - 1)
            res = jnp.zeros((L,), jnp.float32)
            for j in range(L):
                s = jnp.zeros((L,), jnp.float32)
                for k in range(H // L):
                    ks = pl.ds(k * L, L)
                    s = s + ubuf[buf, j, su[j], ks] * ibuf[buf, j, si[j], ks] * wc[k]
                res = jnp.where(lane == j, jnp.sum(s), res)
            outv[pl.ds(r0, L)] = res

        nblk = bpw // L
        issue(0, 0)
        issue(1, 1)

        def pair_body(p, carry):
            c0 = p * 2
            wait(0)
            compute(c0, 0)

            @pl.when(c0 + 2 < nblk)
            def _():
                issue(c0 + 2, 0)

            wait(1)
            compute(c0 + 1, 1)

            @pl.when(c0 + 3 < nblk)
            def _():
                issue(c0 + 3, 1)

            return carry

        lax.fori_loop(0, nblk // 2, pair_body, 0)

        for cp in bias_cps:
            cp.wait()

        def bias_body(g, carry):
            rows = pl.ds(g * L, L)
            outv[rows] = outv[rows] + (ubv[rows] + ibv[rows]) * w_bias + b_lin
            return carry

        lax.fori_loop(0, bpw // L, bias_body, 0)

        pltpu.sync_copy(outv, out_hbm.at[pl.ds(wid * bpw, bpw)])

    return biased_gmf_sc(uids, iids, ue3, ie3, ub_tbl, ib_tbl, wvec)


# speedup vs baseline: 1.0261x; 1.0261x over previous
import functools

import jax
import jax.numpy as jnp
from jax import lax
from jax.experimental import pallas as pl
from jax.experimental.pallas import tpu as pltpu
from jax.experimental.pallas import tpu_sc as plsc

L = 16
SB = 8
RC = 2048


def _tc_relayout_body(x_ref, o_ref):
    o_ref[...] = x_ref[...].T


def _tc_relayout(xt, n, h):
    return pl.pallas_call(
        _tc_relayout_body,
        grid=(pl.cdiv(n, RC),),
        in_specs=[pl.BlockSpec((h, RC), lambda i: (0, i))],
        out_specs=pl.BlockSpec((RC, h), lambda i: (i, 0)),
        out_shape=jax.ShapeDtypeStruct((n, h), jnp.float32),
    )(xt)


def kernel(user_ids, item_ids, user_emb, item_emb, user_bias, item_bias, W, b):
    B = user_ids.shape[0]
    NU = user_emb.shape[0]
    NI = item_emb.shape[0]
    H = user_emb.shape[1]
    info = plsc.get_sparse_core_info()
    NC, NS = info.num_cores, info.num_subcores
    NW = NC * NS
    bpw = B // NW
    CH = 128
    nchk = bpw // CH

    uids = user_ids.astype(jnp.int32)
    iids = item_ids.astype(jnp.int32)
    ue3 = _tc_relayout(user_emb.T, NU, H).reshape(NU // SB, SB, H)
    ie3 = item_emb.reshape(NI // SB, SB, H)
    ub_tbl = user_bias.reshape(-1)
    ib_tbl = item_bias.reshape(-1)
    wvec = jnp.concatenate(
        [W.reshape(-1), b.reshape(-1), jnp.zeros((6,), jnp.float32)]
    )

    mesh = plsc.VectorSubcoreMesh(core_axis_name="c", subcore_axis_name="s")

    @functools.partial(
        pl.kernel,
        out_type=jax.ShapeDtypeStruct((B,), jnp.float32),
        mesh=mesh,
        scratch_types=[
            pltpu.VMEM((bpw,), jnp.int32),
            pltpu.VMEM((bpw,), jnp.int32),
            pltpu.VMEM((2, L, SB, H), jnp.float32),
            pltpu.VMEM((2, L, SB, H), jnp.float32),
            pltpu.VMEM((bpw,), jnp.float32),
            pltpu.VMEM((bpw,), jnp.float32),
            pltpu.VMEM((bpw,), jnp.float32),
            pltpu.VMEM((H + 8,), jnp.float32),
            pltpu.SemaphoreType.DMA,
            pltpu.SemaphoreType.DMA,
            pltpu.SemaphoreType.DMA,
        ],
        compiler_params=pltpu.CompilerParams(
            needs_layout_passes=False, use_tc_tiling_on_sc=True
        ),
    )
    def biased_gmf_sc(uids_hbm, iids_hbm, ue_hbm, ie_hbm, ub_hbm, ib_hbm,
                      w_hbm, out_hbm,
                      ubi, ibi, ubuf, ibuf, ubv, ibv, outv, wv,
                      sem0, sem1, sem2):
        wid = lax.axis_index("s") * NC + lax.axis_index("c")
        ids = pl.ds(wid * bpw, bpw)

        pltpu.sync_copy(uids_hbm.at[ids], ubi)
        pltpu.sync_copy(iids_hbm.at[ids], ibi)
        pltpu.sync_copy(w_hbm, wv)

        bias_cps = []
        for c in range(nchk):
            rows = pl.ds(c * CH, CH)
            bias_cps.append(
                pltpu.async_copy(ub_hbm.at[ubi.at[rows]], ubv.at[rows], sem2))
            bias_cps.append(
                pltpu.async_copy(ib_hbm.at[ibi.at[rows]], ibv.at[rows], sem2))

        sems = (sem0, sem1)

        def issue(c, buf):
            uv = ubi[pl.ds(c * L, L)] >> 3
            iv = ibi[pl.ds(c * L, L)] >> 3
            for j in range(L):
                pltpu.async_copy(ue_hbm.at[uv[j]], ubuf.at[buf, j], sems[buf])
                pltpu.async_copy(ie_hbm.at[iv[j]], ibuf.at[buf, j], sems[buf])

        def wait(buf):
            for j in range(L):
                pltpu.make_async_copy(ue_hbm.at[0], ubuf.at[buf, j],
                                      sems[buf]).wait()
                pltpu.make_async_copy(ie_hbm.at[0], ibuf.at[buf, j],
                                      sems[buf]).wait()

        wc = [wv[pl.ds(k * L, L)] for k in range(H // L)]
        wtail = wv[pl.ds(H - 8, L)]
        w_bias = wtail[8]
        b_lin = wtail[9]
        lane = lax.iota(jnp.int32, L)

        def compute(c, buf):
            r0 = c * L
            su = ubi[pl.ds(r0, L)] & (SB - 1)
            si = ibi[pl.ds(r0, L)] & (SB - 1)
            res = jnp.zeros((L,), jnp.float32)
            for j in range(L):
                s = jnp.zeros((L,), jnp.float32)
                for k in range(H // L):
                    ks = pl.ds(k * L, L)
                    s = s + ubuf[buf, j, su[j], ks] * ibuf[buf, j, si[j], ks] * wc[k]
                res = jnp.where(lane == j, jnp.sum(s), res)
            outv[pl.ds(r0, L)] = res

        nblk = bpw // L
        issue(0, 0)
        issue(1, 1)

        def pair_body(p, carry):
            c0 = p * 2
            wait(0)
            compute(c0, 0)

            @pl.when(c0 + 2 < nblk)
            def _():
                issue(c0 + 2, 0)

            wait(1)
            compute(c0 + 1, 1)

            @pl.when(c0 + 3 < nblk)
            def _():
                issue(c0 + 3, 1)

            return carry

        lax.fori_loop(0, nblk // 2, pair_body, 0)

        for cp in bias_cps:
            cp.wait()

        def bias_body(g, carry):
            rows = pl.ds(g * L, L)
            outv[rows] = outv[rows] + (ubv[rows] + ibv[rows]) * w_bias + b_lin
            return carry

        lax.fori_loop(0, bpw // L, bias_body, 0)

        pltpu.sync_copy(outv, out_hbm.at[pl.ds(wid * bpw, bpw)])

    return biased_gmf_sc(uids, iids, ue3, ie3, ub_tbl, ib_tbl, wvec)


# speedup vs baseline: 1.4089x; 1.3731x over previous
import functools

import jax
import jax.numpy as jnp
from jax import lax
from jax.experimental import pallas as pl
from jax.experimental.pallas import tpu as pltpu
from jax.experimental.pallas import tpu_sc as plsc

L = 16
SB = 8


def kernel(user_ids, item_ids, user_emb, item_emb, user_bias, item_bias, W, b):
    B = user_ids.shape[0]
    NU = user_emb.shape[0]
    NI = item_emb.shape[0]
    H = user_emb.shape[1]
    info = plsc.get_sparse_core_info()
    NC, NS = info.num_cores, info.num_subcores
    NW = NC * NS
    bpw = B // NW
    CH = 128
    nchk = bpw // CH

    uids = user_ids.astype(jnp.int32)
    iids = item_ids.astype(jnp.int32)
    ue3 = user_emb.reshape(NU // SB, SB, H)
    ie3 = item_emb.reshape(NI // SB, SB, H)
    ub_tbl = user_bias.reshape(-1)
    ib_tbl = item_bias.reshape(-1)
    wvec = jnp.concatenate(
        [W.reshape(-1), b.reshape(-1), jnp.zeros((6,), jnp.float32)]
    )

    mesh = plsc.VectorSubcoreMesh(core_axis_name="c", subcore_axis_name="s")

    @functools.partial(
        pl.kernel,
        out_type=jax.ShapeDtypeStruct((B,), jnp.float32),
        mesh=mesh,
        scratch_types=[
            pltpu.VMEM((bpw,), jnp.int32),
            pltpu.VMEM((bpw,), jnp.int32),
            pltpu.VMEM((2, L, SB, H), jnp.float32),
            pltpu.VMEM((2, L, SB, H), jnp.float32),
            pltpu.VMEM((bpw,), jnp.float32),
            pltpu.VMEM((bpw,), jnp.float32),
            pltpu.VMEM((bpw,), jnp.float32),
            pltpu.VMEM((H + 8,), jnp.float32),
            pltpu.SemaphoreType.DMA,
            pltpu.SemaphoreType.DMA,
            pltpu.SemaphoreType.DMA,
        ],
        compiler_params=pltpu.CompilerParams(
            needs_layout_passes=False, use_tc_tiling_on_sc=True
        ),
    )
    def biased_gmf_sc(uids_hbm, iids_hbm, ue_hbm, ie_hbm, ub_hbm, ib_hbm,
                      w_hbm, out_hbm,
                      ubi, ibi, ubuf, ibuf, ubv, ibv, outv, wv,
                      sem0, sem1, sem2):
        wid = lax.axis_index("s") * NC + lax.axis_index("c")
        ids = pl.ds(wid * bpw, bpw)

        pltpu.sync_copy(uids_hbm.at[ids], ubi)
        pltpu.sync_copy(iids_hbm.at[ids], ibi)
        pltpu.sync_copy(w_hbm, wv)

        bias_cps = []
        for c in range(nchk):
            rows = pl.ds(c * CH, CH)
            bias_cps.append(
                pltpu.async_copy(ub_hbm.at[ubi.at[rows]], ubv.at[rows], sem2))
            bias_cps.append(
                pltpu.async_copy(ib_hbm.at[ibi.at[rows]], ibv.at[rows], sem2))

        sems = (sem0, sem1)

        def issue(c, buf):
            uv = ubi[pl.ds(c * L, L)] >> 3
            iv = ibi[pl.ds(c * L, L)] >> 3
            for j in range(L):
                pltpu.async_copy(ue_hbm.at[uv[j]], ubuf.at[buf, j], sems[buf])
                pltpu.async_copy(ie_hbm.at[iv[j]], ibuf.at[buf, j], sems[buf])

        def wait(buf):
            for j in range(L):
                pltpu.make_async_copy(ue_hbm.at[0], ubuf.at[buf, j],
                                      sems[buf]).wait()
                pltpu.make_async_copy(ie_hbm.at[0], ibuf.at[buf, j],
                                      sems[buf]).wait()

        wc = [wv[pl.ds(k * L, L)] for k in range(H // L)]
        wtail = wv[pl.ds(H - 8, L)]
        w_bias = wtail[8]
        b_lin = wtail[9]
        lane = lax.iota(jnp.int32, L)

        def compute(c, buf):
            r0 = c * L
            su = ubi[pl.ds(r0, L)] & (SB - 1)
            si = ibi[pl.ds(r0, L)] & (SB - 1)
            res = jnp.zeros((L,), jnp.float32)
            for j in range(L):
                s = jnp.zeros((L,), jnp.float32)
                for k in range(H // L):
                    ks = pl.ds(k * L, L)
                    s = s + ubuf[buf, j, su[j], ks] * ibuf[buf, j, si[j], ks] * wc[k]
                res = jnp.where(lane == j, jnp.sum(s), res)
            outv[pl.ds(r0, L)] = res

        nblk = bpw // L
        issue(0, 0)
        issue(1, 1)

        def pair_body(p, carry):
            c0 = p * 2
            wait(0)
            compute(c0, 0)

            @pl.when(c0 + 2 < nblk)
            def _():
                issue(c0 + 2, 0)

            wait(1)
            compute(c0 + 1, 1)

            @pl.when(c0 + 3 < nblk)
            def _():
                issue(c0 + 3, 1)

            return carry

        lax.fori_loop(0, nblk // 2, pair_body, 0)

        for cp in bias_cps:
            cp.wait()

        def bias_body(g, carry):
            rows = pl.ds(g * L, L)
            outv[rows] = outv[rows] + (ubv[rows] + ibv[rows]) * w_bias + b_lin
            return carry

        lax.fori_loop(0, bpw // L, bias_body, 0)

        pltpu.sync_copy(outv, out_hbm.at[pl.ds(wid * bpw, bpw)])

    return biased_gmf_sc(uids, iids, ue3, ie3, ub_tbl, ib_tbl, wvec)
